# baseline (speedup 1.0000x reference)
import functools

import jax
import jax.numpy as jnp
import numpy as np
from jax import lax
from jax.experimental import pallas as pl
from jax.experimental.pallas import tpu as pltpu
from jax.experimental.pallas import tpu_sc as plsc

B, T, C_IN = 1024, 200, 131
C_OUT = 84

_COLS = np.array(
    [i for i in range(3, 66) if (i - 3) % 3 != 2]
    + [i for i in range(68, 131) if (i - 68) % 3 != 2],
    dtype=np.int32,
)
assert _COLS.shape[0] == C_OUT

GROUP_OUT = 336
N_PAT = GROUP_OUT // 16
_P = np.arange(GROUP_OUT, dtype=np.int32)
_IDX_NP = np.concatenate([
    _P // C_OUT,
    _COLS[_P % C_OUT],
    _P % C_OUT,
])

NC = 2
NS = 16
NW = NC * NS
B_PER_W = B // NW
GROUPS = T // 4
PAIRS = B_PER_W // 2


def _sc_reduce(x, idx):
    mesh = plsc.VectorSubcoreMesh(core_axis_name="c", subcore_axis_name="s")

    @functools.partial(
        pl.kernel,
        mesh=mesh,
        out_type=jax.ShapeDtypeStruct((B, T, C_OUT), jnp.float32),
        scratch_types=[
            pltpu.VMEM((3 * GROUP_OUT,), jnp.int32),
            pltpu.VMEM((T, C_IN), jnp.float32),
            pltpu.VMEM((T, C_IN), jnp.float32),
            pltpu.VMEM((T, C_OUT), jnp.float32),
            pltpu.VMEM((T, C_OUT), jnp.float32),
            pltpu.SemaphoreType.DMA,
            pltpu.SemaphoreType.DMA,
            pltpu.SemaphoreType.DMA,
            pltpu.SemaphoreType.DMA,
        ],
        compiler_params=pltpu.CompilerParams(
            needs_layout_passes=False, use_tc_tiling_on_sc=False),
    )
    def k(x_hbm, idx_hbm, out_hbm, idx_v, in0, in1, out0, out1,
          sin0, sin1, sout0, sout1):
        wid = lax.axis_index("s") * NC + lax.axis_index("c")
        b0 = wid * B_PER_W
        pltpu.sync_copy(idx_hbm, idx_v)
        rpat = [idx_v[pl.ds(j * 16, 16)] for j in range(N_PAT)]
        cin = [idx_v[pl.ds(GROUP_OUT + j * 16, 16)] for j in range(N_PAT)]
        cout = [idx_v[pl.ds(2 * GROUP_OUT + j * 16, 16)]
                for j in range(N_PAT)]

        def start_in(s, buf, sem):
            s = jnp.minimum(s, B_PER_W - 1)
            pltpu.async_copy(x_hbm.at[b0 + s], buf, sem)

        def wait_in(buf, sem):
            pltpu.make_async_copy(x_hbm.at[0], buf, sem).wait()

        def start_out(buf, s, sem):
            pltpu.async_copy(buf, out_hbm.at[b0 + s], sem)

        def wait_out(buf, sem):
            pltpu.make_async_copy(buf, out_hbm.at[0], sem).wait()

        def compute(in_ref, out_ref):
            def grp(g, c):
                rbase = jnp.full((16,), 4 * g, jnp.int32)
                for j in range(N_PAT):
                    r = rpat[j] + rbase
                    v = plsc.load_gather(in_ref, [r, cin[j]])
                    plsc.store_scatter(out_ref, [r, cout[j]], v)
                return c
            lax.fori_loop(0, GROUPS, grp, 0)

        start_in(0, in0, sin0)
        start_in(1, in1, sin1)
        wait_in(in0, sin0)
        compute(in0, out0)
        start_out(out0, 0, sout0)
        start_in(2, in0, sin0)
        wait_in(in1, sin1)
        compute(in1, out1)
        start_out(out1, 1, sout1)
        start_in(3, in1, sin1)

        def pair(t, c):
            s0 = 2 * t
            wait_in(in0, sin0)
            wait_out(out0, sout0)
            compute(in0, out0)
            start_out(out0, s0, sout0)
            start_in(s0 + 2, in0, sin0)
            wait_in(in1, sin1)
            wait_out(out1, sout1)
            compute(in1, out1)
            start_out(out1, s0 + 1, sout1)
            start_in(s0 + 3, in1, sin1)
            return c

        lax.fori_loop(1, PAIRS, pair, 0)

        wait_in(in0, sin0)
        wait_in(in1, sin1)
        wait_out(out0, sout0)
        wait_out(out1, sout1)

    return k(x, idx)


def kernel(X):
    idx = jnp.asarray(_IDX_NP)
    return _sc_reduce(X, idx)


# speedup vs baseline: 1.1850x; 1.1850x over previous
import functools

import jax
import jax.numpy as jnp
import numpy as np
from jax import lax
from jax.experimental import pallas as pl
from jax.experimental.pallas import tpu as pltpu
from jax.experimental.pallas import tpu_sc as plsc

B, T, C_IN = 1024, 200, 131
C_OUT = 84
ROWS = B * T
LANES = 128
IN_LINES = ROWS * C_IN // LANES
OUT_LINES = ROWS * C_OUT // LANES

_COLS = np.array(
    [i for i in range(3, 66) if (i - 3) % 3 != 2]
    + [i for i in range(68, 131) if (i - 68) % 3 != 2],
    dtype=np.int32,
)
assert _COLS.shape[0] == C_OUT

GROUP_OUT = 336
N_PAT = GROUP_OUT // 16
_P = np.arange(GROUP_OUT, dtype=np.int32)
_IDX_NP = (_P // C_OUT) * C_IN + _COLS[_P % C_OUT]

NC = 2
NS = 16
NW = NC * NS
CH_FRAMES = 128
IN_CH_LINES = 131
OUT_CH_LINES = 84
IN_CHUNK = CH_FRAMES * C_IN
OUT_CHUNK = CH_FRAMES * C_OUT
STEPS = ROWS // (NW * CH_FRAMES)
PAIRS = STEPS // 2
GROUPS = CH_FRAMES // 4


def _sc_reduce(x2, idx):
    mesh = plsc.VectorSubcoreMesh(core_axis_name="c", subcore_axis_name="s")

    @functools.partial(
        pl.kernel,
        mesh=mesh,
        out_type=jax.ShapeDtypeStruct((OUT_LINES, LANES), jnp.float32),
        scratch_types=[
            pltpu.VMEM((GROUP_OUT,), jnp.int32),
            pltpu.VMEM((IN_CH_LINES, LANES), jnp.float32),
            pltpu.VMEM((IN_CH_LINES, LANES), jnp.float32),
            pltpu.VMEM((OUT_CH_LINES, LANES), jnp.float32),
            pltpu.VMEM((OUT_CH_LINES, LANES), jnp.float32),
            pltpu.SemaphoreType.DMA,
            pltpu.SemaphoreType.DMA,
            pltpu.SemaphoreType.DMA,
            pltpu.SemaphoreType.DMA,
        ],
        compiler_params=pltpu.CompilerParams(
            needs_layout_passes=False, use_tc_tiling_on_sc=False),
    )
    def k(x_hbm, idx_hbm, out_hbm, idx_v, in0, in1, out0, out1,
          sin0, sin1, sout0, sout1):
        wid = lax.axis_index("s") * NC + lax.axis_index("c")
        in_base = wid * (STEPS * IN_CH_LINES)
        out_base = wid * (STEPS * OUT_CH_LINES)
        pltpu.sync_copy(idx_hbm, idx_v)
        pats = [idx_v[pl.ds(j * 16, 16)] for j in range(N_PAT)]

        def start_in(s, buf, sem):
            s = jnp.minimum(s, STEPS - 1)
            pltpu.async_copy(
                x_hbm.at[pl.ds(in_base + s * IN_CH_LINES, IN_CH_LINES)],
                buf, sem)

        def wait_in(buf, sem):
            pltpu.make_async_copy(
                x_hbm.at[pl.ds(0, IN_CH_LINES)], buf, sem).wait()

        def start_out(buf, s, sem):
            pltpu.async_copy(
                buf,
                out_hbm.at[pl.ds(out_base + s * OUT_CH_LINES, OUT_CH_LINES)],
                sem)

        def wait_out(buf, sem):
            pltpu.make_async_copy(
                buf, out_hbm.at[pl.ds(0, OUT_CH_LINES)], sem).wait()

        def compute(in_ref, out_ref):
            def grp(g, c):
                base = jnp.full((16,), g * (4 * C_IN), jnp.int32)
                q0 = g * GROUP_OUT
                for j in range(N_PAT):
                    f = pats[j] + base
                    line = lax.shift_right_logical(f, 7)
                    lane = lax.bitwise_and(f, 127)
                    v = plsc.load_gather(in_ref, [line, lane])
                    q = q0 + j * 16
                    out_ref[lax.shift_right_logical(q, 7),
                            pl.ds(lax.bitwise_and(q, 127), 16)] = v
                return c
            lax.fori_loop(0, GROUPS, grp, 0)

        start_in(0, in0, sin0)
        start_in(1, in1, sin1)
        wait_in(in0, sin0)
        compute(in0, out0)
        start_out(out0, 0, sout0)
        start_in(2, in0, sin0)
        wait_in(in1, sin1)
        compute(in1, out1)
        start_out(out1, 1, sout1)
        start_in(3, in1, sin1)

        def pair(t, c):
            s0 = 2 * t
            wait_in(in0, sin0)
            wait_out(out0, sout0)
            compute(in0, out0)
            start_out(out0, s0, sout0)
            start_in(s0 + 2, in0, sin0)
            wait_in(in1, sin1)
            wait_out(out1, sout1)
            compute(in1, out1)
            start_out(out1, s0 + 1, sout1)
            start_in(s0 + 3, in1, sin1)
            return c

        lax.fori_loop(1, PAIRS, pair, 0)

        wait_in(in0, sin0)
        wait_in(in1, sin1)
        wait_out(out0, sout0)
        wait_out(out1, sout1)

    return k(x2, idx)


def kernel(X):
    x2 = X.reshape(IN_LINES, LANES)
    idx = jnp.asarray(_IDX_NP)
    out2 = _sc_reduce(x2, idx)
    return out2.reshape(B, T, C_OUT)


# speedup vs baseline: 2.5010x; 2.1105x over previous
import jax
import jax.numpy as jnp
import numpy as np
from jax.experimental import pallas as pl
from jax.experimental.pallas import tpu as pltpu

B, T, C_IN = 1024, 200, 131
C_OUT = 84
ROWS = B * T

_COLS = np.array(
    [i for i in range(3, 66) if (i - 3) % 3 != 2]
    + [i for i in range(68, 131) if (i - 68) % 3 != 2],
    dtype=np.int32,
)
assert _COLS.shape[0] == C_OUT

_SEL_NP = np.zeros((C_IN, C_OUT), dtype=np.float32)
_SEL_NP[_COLS, np.arange(C_OUT)] = 1.0

BLK = 2048
GRID = ROWS // BLK


def _body(x_ref, s_ref, o_ref):
    o_ref[...] = jnp.dot(
        x_ref[...], s_ref[...], preferred_element_type=jnp.float32)


def kernel(X):
    x2 = X.reshape(ROWS, C_IN)
    sel = jnp.asarray(_SEL_NP)
    out = pl.pallas_call(
        _body,
        grid=(GRID,),
        in_specs=[
            pl.BlockSpec((BLK, C_IN), lambda i: (i, 0)),
            pl.BlockSpec((C_IN, C_OUT), lambda i: (0, 0)),
        ],
        out_specs=pl.BlockSpec((BLK, C_OUT), lambda i: (i, 0)),
        out_shape=jax.ShapeDtypeStruct((ROWS, C_OUT), jnp.float32),
        compiler_params=pltpu.CompilerParams(
            dimension_semantics=("arbitrary",)),
    )(x2, sel)
    return out.reshape(B, T, C_OUT)
